# baseline (speedup 1.0000x reference)
import jax
import jax.numpy as jnp
from jax import lax
from jax.experimental import pallas as pl
from jax.experimental.pallas import tpu as pltpu

_MARGIN = 0.1
_ALPHA = 0.5
_BETA = 0.5
_C = 32
_N = 1024
_D = 2048
_KSLAB = 256
_NSLAB = _D // _KSLAB
_CHUNK = 16
_NEG_INF = float('-inf')
_POS_INF = float('inf')


def _nt(a, b):
    return lax.dot_general(a, b, (((1,), (1,)), ((), ())),
                           preferred_element_type=jnp.float32)


def _body(f_slab_ref, tcol_ref, trow_ref, out_ref,
          g_scr, sqc_scr, sqr_scr, cen_scr):
    s = pl.program_id(0)
    t_col = tcol_ref[...]
    t_row = trow_ref[...]

    @pl.when(s < _NSLAB)
    def _slab_step():
        slab = f_slab_ref[...]
        ff = slab * slab
        sqc_part = jnp.sum(ff, axis=1, keepdims=True)
        ones_k = jnp.ones((1, _KSLAB), jnp.float32)
        sqr_part = _nt(ones_k, ff)
        sb = slab.astype(jnp.bfloat16)
        prod = _nt(sb, sb)
        labels_row = lax.broadcasted_iota(jnp.int32, (1, _C), 1)
        onehot_nc = (t_col == labels_row).astype(jnp.float32)
        cen_part = lax.dot_general(onehot_nc, slab, (((0,), (0,)), ((), ())),
                                   preferred_element_type=jnp.float32)
        cen_scr[:, pl.ds(s * _KSLAB, _KSLAB)] = cen_part

        @pl.when(s == 0)
        def _():
            g_scr[...] = prod
            sqc_scr[...] = sqc_part
            sqr_scr[...] = sqr_part

        @pl.when(s > 0)
        def _():
            g_scr[...] += prod
            sqc_scr[...] += sqc_part
            sqr_scr[...] += sqr_part

    @pl.when(s == _NSLAB)
    def _finalize():
        sq_row = sqr_scr[...]

        def chunk_body(r, acc):
            g8 = g_scr[pl.ds(r * _CHUNK, _CHUNK), :]
            t8 = tcol_ref[pl.ds(r * _CHUNK, _CHUNK), :]
            sq8 = sqc_scr[pl.ds(r * _CHUNK, _CHUNK), :]
            vals = jnp.where(t8 == t_row, sq8 - 2.0 * g8, _NEG_INF)
            return jnp.maximum(acc, jnp.max(vals, axis=0, keepdims=True))

        acc0 = jnp.full((1, _N), _NEG_INF, jnp.float32)
        colmax = lax.fori_loop(0, _N // _CHUNK, chunk_body, acc0)
        colmax_dsq = colmax + sq_row

        labels_col = lax.broadcasted_iota(jnp.int32, (_C, 1), 0)
        cmask = labels_col == t_row
        cmax_dsq = jnp.max(jnp.where(cmask, colmax_dsq, _NEG_INF),
                           axis=1, keepdims=True)
        cmax = jnp.sqrt(jnp.clip(cmax_dsq, 1e-12, None))
        counts_col = jnp.sum(cmask.astype(jnp.float32), axis=1,
                             keepdims=True)
        contrib = jnp.where(counts_col >= 2.0, 1.0 / cmax, 0.0)
        intra = jnp.sum(contrib)

        centers = cen_scr[...] / jnp.maximum(counts_col, 1.0)
        cc = centers * centers
        csq_col = jnp.sum(cc, axis=1, keepdims=True)
        csq_row = _nt(jnp.ones((1, _D), jnp.float32), cc)
        gc = _nt(centers, centers)
        dc = jnp.sqrt(jnp.clip(csq_col + csq_row - 2.0 * gc, 1e-12, None))
        labels_row = lax.broadcasted_iota(jnp.int32, (1, _C), 1)
        onehot_nc = (t_col == labels_row).astype(jnp.float32)
        counts_row = lax.dot_general(jnp.ones((1, _N), jnp.float32), onehot_nc,
                                     (((1,), (0,)), ((), ())),
                                     preferred_element_type=jnp.float32)
        valid = (counts_col > 0.0) & (counts_row > 0.0) & (dc > 0.0)
        min_inter = jnp.min(jnp.where(valid, dc, _POS_INF))

        out_ref[0, 0] = _ALPHA * (_MARGIN - min_inter) + _BETA * intra


def kernel(features, targets):
    t_col = targets.reshape(_N, 1).astype(jnp.int32)
    t_row = targets.reshape(1, _N).astype(jnp.int32)
    out = pl.pallas_call(
        _body,
        grid=(_NSLAB + 1,),
        in_specs=[
            pl.BlockSpec((_N, _KSLAB),
                         lambda s: (0, jnp.minimum(s, _NSLAB - 1))),
            pl.BlockSpec((_N, 1), lambda s: (0, 0)),
            pl.BlockSpec((1, _N), lambda s: (0, 0)),
        ],
        out_specs=pl.BlockSpec(memory_space=pltpu.SMEM),
        out_shape=jax.ShapeDtypeStruct((1, 1), jnp.float32),
        scratch_shapes=[
            pltpu.VMEM((_N, _N), jnp.float32),
            pltpu.VMEM((_N, 1), jnp.float32),
            pltpu.VMEM((1, _N), jnp.float32),
            pltpu.VMEM((_C, _D), jnp.float32),
        ],
    )(features, t_col, t_row)
    return out[0, 0]


# speedup vs baseline: 1.1228x; 1.1228x over previous
import jax
import jax.numpy as jnp
from jax import lax
from jax.experimental import pallas as pl
from jax.experimental.pallas import tpu as pltpu

_MARGIN = 0.1
_ALPHA = 0.5
_BETA = 0.5
_C = 32
_N = 1024
_D = 2048
_STRIP = 128
_NEG_INF = float('-inf')
_POS_INF = float('inf')


def _nt(a, b):
    return lax.dot_general(a, b, (((1,), (1,)), ((), ())),
                           preferred_element_type=jnp.float32)


def _body(f_ref, tcol_ref, trow_ref, out_ref):
    f = f_ref[...]
    t_col = tcol_ref[...]
    t_row = trow_ref[...]

    ff = f * f
    sq_col = jnp.sum(ff, axis=1, keepdims=True)
    sq_row = _nt(jnp.ones((1, _D), jnp.float32), ff)
    fb = f.astype(jnp.bfloat16)

    colmax = jnp.full((1, _N), _NEG_INF, jnp.float32)
    for i in range(_N // _STRIP):
        sl = slice(i * _STRIP, (i + 1) * _STRIP)
        prod = _nt(fb[sl, :], fb)
        vals = jnp.where(t_col[sl, :] == t_row,
                         sq_col[sl, :] - 2.0 * prod, _NEG_INF)
        colmax = jnp.maximum(colmax, jnp.max(vals, axis=0, keepdims=True))
    colmax_dsq = colmax + sq_row

    labels_col = lax.broadcasted_iota(jnp.int32, (_C, 1), 0)
    cmask = labels_col == t_row
    cmax_dsq = jnp.max(jnp.where(cmask, colmax_dsq, _NEG_INF),
                       axis=1, keepdims=True)
    cmax = jnp.sqrt(jnp.clip(cmax_dsq, 1e-12, None))
    counts_col = jnp.sum(cmask.astype(jnp.float32), axis=1,
                         keepdims=True)
    contrib = jnp.where(counts_col >= 2.0, 1.0 / cmax, 0.0)
    intra = jnp.sum(contrib)

    labels_row = lax.broadcasted_iota(jnp.int32, (1, _C), 1)
    onehot_nc = (t_col == labels_row).astype(jnp.float32)
    centers = lax.dot_general(onehot_nc, f, (((0,), (0,)), ((), ())),
                              preferred_element_type=jnp.float32)
    centers = centers / jnp.maximum(counts_col, 1.0)
    cc = centers * centers
    csq_col = jnp.sum(cc, axis=1, keepdims=True)
    csq_row = _nt(jnp.ones((1, _D), jnp.float32), cc)
    gc = _nt(centers, centers)
    dc = jnp.sqrt(jnp.clip(csq_col + csq_row - 2.0 * gc, 1e-12, None))
    counts_row = lax.dot_general(jnp.ones((1, _N), jnp.float32), onehot_nc,
                                 (((1,), (0,)), ((), ())),
                                 preferred_element_type=jnp.float32)
    valid = (counts_col > 0.0) & (counts_row > 0.0) & (dc > 0.0)
    min_inter = jnp.min(jnp.where(valid, dc, _POS_INF))

    out_ref[0, 0] = _ALPHA * (_MARGIN - min_inter) + _BETA * intra


def kernel(features, targets):
    t_col = targets.reshape(_N, 1).astype(jnp.int32)
    t_row = targets.reshape(1, _N).astype(jnp.int32)
    out = pl.pallas_call(
        _body,
        out_specs=pl.BlockSpec(memory_space=pltpu.SMEM),
        out_shape=jax.ShapeDtypeStruct((1, 1), jnp.float32),
    )(features, t_col, t_row)
    return out[0, 0]


# speedup vs baseline: 1.7356x; 1.5458x over previous
import jax
import jax.numpy as jnp
from jax import lax
from jax.experimental import pallas as pl
from jax.experimental.pallas import tpu as pltpu

_MARGIN = 0.1
_ALPHA = 0.5
_BETA = 0.5
_C = 32
_N = 1024
_D = 2048
_NEG_INF = float('-inf')
_POS_INF = float('inf')


def _nt(a, b):
    return lax.dot_general(a, b, (((1,), (1,)), ((), ())),
                           preferred_element_type=jnp.float32)


def _body(f_ref, tcol_ref, trow_ref, out_ref):
    f = f_ref[...]
    t_col = tcol_ref[...]
    t_row = trow_ref[...]

    ones_row = jnp.ones((1, _D), jnp.float32)
    ff = f * f
    sq_col = _nt(ff, ones_row)
    sq_row = _nt(ones_row, ff)
    g = _nt(f, f)

    halfsq = 0.5 * sq_col
    vals = jnp.where(t_col == t_row, halfsq - g, _NEG_INF)
    colmax2 = jnp.max(vals, axis=0, keepdims=True)
    colmax_dsq = 2.0 * colmax2 + sq_row

    labels_col = lax.broadcasted_iota(jnp.int32, (_C, 1), 0)
    cmask = labels_col == t_row
    cmax_dsq = jnp.max(jnp.where(cmask, colmax_dsq, _NEG_INF),
                       axis=1, keepdims=True)
    cmax = jnp.sqrt(jnp.clip(cmax_dsq, 1e-12, None))
    counts_col = jnp.sum(cmask.astype(jnp.float32), axis=1,
                         keepdims=True)
    contrib = jnp.where(counts_col >= 2.0, 1.0 / cmax, 0.0)
    intra = jnp.sum(contrib)

    labels_row = lax.broadcasted_iota(jnp.int32, (1, _C), 1)
    onehot_nc = (t_col == labels_row).astype(jnp.float32)
    centers = lax.dot_general(onehot_nc, f, (((0,), (0,)), ((), ())),
                              preferred_element_type=jnp.float32)
    centers = centers / jnp.maximum(counts_col, 1.0)
    cc = centers * centers
    csq_col = jnp.sum(cc, axis=1, keepdims=True)
    csq_row = _nt(jnp.ones((1, _D), jnp.float32), cc)
    gc = _nt(centers, centers)
    dc = jnp.sqrt(jnp.clip(csq_col + csq_row - 2.0 * gc, 1e-12, None))
    counts_row = lax.dot_general(jnp.ones((1, _N), jnp.float32), onehot_nc,
                                 (((1,), (0,)), ((), ())),
                                 preferred_element_type=jnp.float32)
    valid = (counts_col > 0.0) & (counts_row > 0.0) & (dc > 0.0)
    min_inter = jnp.min(jnp.where(valid, dc, _POS_INF))

    out_ref[0, 0] = _ALPHA * (_MARGIN - min_inter) + _BETA * intra


def kernel(features, targets):
    t_col = targets.reshape(_N, 1).astype(jnp.int32)
    t_row = targets.reshape(1, _N).astype(jnp.int32)
    out = pl.pallas_call(
        _body,
        out_specs=pl.BlockSpec(memory_space=pltpu.SMEM),
        out_shape=jax.ShapeDtypeStruct((1, 1), jnp.float32),
    )(features, t_col, t_row)
    return out[0, 0]
